# baseline (speedup 1.0000x reference)
import functools

import jax
import jax.numpy as jnp
from jax import lax
from jax.experimental import pallas as pl
from jax.experimental.pallas import tpu as pltpu
from jax.experimental.pallas import tpu_sc as plsc

_C = 1000
_B = 16384
_S = 0.1
_COEF_B = _S / (_C - 1)
_COEF_A = 1.0 - _S - _COEF_B

_B_SC = 4096
_B_TC = _B - _B_SC

_BLOCK_ROWS = 512
_NW = 32
_ROWS_PER_W = _B_SC // _NW
_GRP = 16
_NGRP = _ROWS_PER_W // _GRP
_LANES = 16
_NCHUNK = _C // _LANES
_TAIL = _C - _NCHUNK * _LANES

_LN2 = 0.6931471805599453


def _tc_body(x_ref, t_ref, out_ref):
    x = x_ref[...]
    t = t_ref[...]
    m = jnp.max(x, axis=1, keepdims=True)
    e = jnp.exp(x - m)
    s = jnp.sum(e, axis=1, keepdims=True)
    lse = m + jnp.log(s)
    rowsum = jnp.sum(x, axis=1, keepdims=True)
    cols = jax.lax.broadcasted_iota(jnp.int32, x.shape, 1)
    tgt = jnp.sum(jnp.where(cols == t, x, 0.0), axis=1, keepdims=True)
    part = lse - _COEF_A * tgt - _COEF_B * rowsum
    out_ref[0] = jnp.sum(part, axis=0, keepdims=True)


def _ln_vec(s):
    bits = lax.bitcast_convert_type(s, jnp.int32)
    e = ((bits >> 23) & 0xFF) - 127
    mbits = (bits & 0x7FFFFF) | 0x3F800000
    m = lax.bitcast_convert_type(mbits, jnp.float32)
    t = (m - 1.0) / (m + 1.0)
    t2 = t * t
    p = 1.0 / 9.0
    p = p * t2 + 1.0 / 7.0
    p = p * t2 + 1.0 / 5.0
    p = p * t2 + 1.0 / 3.0
    p = p * t2 + 1.0
    return e.astype(jnp.float32) * _LN2 + 2.0 * t * p


_UNROLL = 8
_NSTEP = _C // _UNROLL


def _sc_body(logits_hbm, target_hbm, out_hbm, rows_ref, tgt_ref, acc_ref):
    wid = lax.axis_index("s") * 2 + lax.axis_index("c")
    row0 = _B_TC + wid * _ROWS_PER_W
    lane = lax.iota(jnp.int32, _LANES)
    zero = jnp.zeros((_LANES,), jnp.float32)
    zi = jnp.zeros((_LANES,), jnp.int32)

    pltpu.sync_copy(target_hbm.at[pl.ds(row0, _ROWS_PER_W)], tgt_ref)

    def group(g, acc):
        pltpu.sync_copy(logits_hbm.at[pl.ds(row0 + g * _GRP, _GRP)], rows_ref)

        def p1(_, c):
            mv, rv, ci = c
            for k in range(_UNROLL):
                v = plsc.load_gather(rows_ref, [lane, ci + k])
                mv = jnp.maximum(mv, v)
                rv = rv + v
            return (mv, rv, ci + _UNROLL)

        mv, rv, _ = lax.fori_loop(
            0, _NSTEP, p1, (jnp.full((_LANES,), -3.0e38, jnp.float32), zero, zi)
        )

        def p2(_, c):
            sv, ci = c
            for k in range(_UNROLL):
                v = plsc.load_gather(rows_ref, [lane, ci + k])
                sv = sv + jnp.exp(v - mv)
            return (sv, ci + _UNROLL)

        sv, _ = lax.fori_loop(0, _NSTEP, p2, (zero, zi))
        tvec = tgt_ref[pl.ds(g * _GRP, _GRP)]
        tg = plsc.load_gather(rows_ref, [lane, tvec])
        lse = mv + _ln_vec(sv)
        return acc + (lse - _COEF_A * tg - _COEF_B * rv)

    acc = lax.fori_loop(0, _NGRP, group, zero)
    acc_ref[...] = acc
    pltpu.sync_copy(acc_ref, out_hbm.at[wid])


@functools.partial(jax.jit, static_argnames=("interpret",))
def _loss(logits, target, interpret=False):
    t2d = target.reshape(_B, 1)

    sc_call = pl.kernel(
        _sc_body,
        mesh=plsc.VectorSubcoreMesh(core_axis_name="c", subcore_axis_name="s"),
        out_type=jax.ShapeDtypeStruct((_NW, _LANES), jnp.float32),
        scratch_types=[
            pltpu.VMEM((_GRP, _C), jnp.float32),
            pltpu.VMEM((_ROWS_PER_W,), jnp.int32),
            pltpu.VMEM((_LANES,), jnp.float32),
        ],
        compiler_params=pltpu.CompilerParams(needs_layout_passes=False),
    )
    sc_part = sc_call(logits, target)

    grid = _B_TC // _BLOCK_ROWS
    tc_part = pl.pallas_call(
        _tc_body,
        grid=(grid,),
        in_specs=[
            pl.BlockSpec((_BLOCK_ROWS, _C), lambda i: (i, 0)),
            pl.BlockSpec((_BLOCK_ROWS, 1), lambda i: (i, 0)),
        ],
        out_specs=pl.BlockSpec((1, 1, 1), lambda i: (i, 0, 0)),
        out_shape=jax.ShapeDtypeStruct((grid, 1, 1), jnp.float32),
        compiler_params=pltpu.CompilerParams(
            dimension_semantics=("parallel",),
        ),
        interpret=interpret,
    )(logits, t2d)

    return (jnp.sum(tc_part) + jnp.sum(sc_part)) * (1.0 / _B)


def kernel(logits, target):
    return _loss(logits, target)
